# baseline (speedup 1.0000x reference)
import jax
import jax.numpy as jnp
from jax.experimental import pallas as pl
import jax.experimental.pallas.tpu as pltpu

SEQ_LEN = 2048
D_MODEL = 1024
N_REP = 4
BS = 256


def _body(emb_ref, out_hbm, sems):
    i = pl.program_id(0)
    copies = [
        pltpu.make_async_copy(
            emb_ref,
            out_hbm.at[pl.ds(i * BS, BS), j, :],
            sems.at[j],
        )
        for j in range(N_REP)
    ]
    for c in copies:
        c.start()
    for c in copies:
        c.wait()


def kernel(z, pos_embed):
    out = pl.pallas_call(
        _body,
        grid=(SEQ_LEN // BS,),
        in_specs=[pl.BlockSpec((BS, D_MODEL), lambda i: (i, 0))],
        out_specs=pl.BlockSpec(memory_space=pltpu.MemorySpace.HBM),
        out_shape=jax.ShapeDtypeStruct((SEQ_LEN, N_REP, D_MODEL), z.dtype),
        scratch_shapes=[pltpu.SemaphoreType.DMA((N_REP,))],
    )(pos_embed)
    return out


# speedup vs baseline: 1.3009x; 1.3009x over previous
import jax
import jax.numpy as jnp
from jax.experimental import pallas as pl
import jax.experimental.pallas.tpu as pltpu

SEQ_LEN = 2048
D_MODEL = 1024
N_REP = 4
BS = 1024


def _body(emb_ref, out_hbm, sems):
    i = pl.program_id(0)
    copies = [
        pltpu.make_async_copy(
            emb_ref,
            out_hbm.at[pl.ds(i * BS, BS), j, :],
            sems.at[j],
        )
        for j in range(N_REP)
    ]
    for c in copies:
        c.start()
    for c in copies:
        c.wait()


def kernel(z, pos_embed):
    out = pl.pallas_call(
        _body,
        grid=(SEQ_LEN // BS,),
        in_specs=[pl.BlockSpec((BS, D_MODEL), lambda i: (i, 0))],
        out_specs=pl.BlockSpec(memory_space=pltpu.MemorySpace.HBM),
        out_shape=jax.ShapeDtypeStruct((SEQ_LEN, N_REP, D_MODEL), z.dtype),
        scratch_shapes=[pltpu.SemaphoreType.DMA((N_REP,))],
    )(pos_embed)
    return out
